# baseline (speedup 1.0000x reference)
import functools

import jax
import jax.numpy as jnp
from jax import lax
from jax.experimental import pallas as pl
from jax.experimental.pallas import tpu as pltpu
from jax.experimental.pallas import tpu_sc as plsc

TOPK_NUM = 64
TEMP = 0.1
EPS = 1e-20

_NC = 2
_NS = 16
_L = 16


def _stage_a_body(x_ref, w_ref, ut_ref, zt_ref, xt_ref, mp_ref):
    W = w_ref[...]
    n, c, hw = x_ref.shape
    cp = xt_ref.shape[2]
    for b in range(n):
        x = x_ref[b]
        ut = ut_ref[b]
        xt = jnp.transpose(x)
        mt = lax.dot_general(
            xt, W, (((1,), (1,)), ((), ())),
            preferred_element_type=jnp.float32,
        )
        g = -jnp.log(-jnp.log(ut + EPS) + EPS)
        zt_ref[b] = mt + g
        xt_ref[b] = jnp.concatenate(
            [xt, jnp.zeros((hw, cp - c), jnp.float32)], axis=1
        )
        mmax = jnp.max(mt, axis=1, keepdims=True)
        e = jnp.exp(mt - mmax)
        mpT = e / jnp.sum(e, axis=1, keepdims=True)
        mp_ref[b] = jnp.transpose(mpT)


def _stage_c_body(mp_ref, candv_ref, xcand_ref, out_ref):
    n, c, hw = out_ref.shape
    qn = candv_ref.shape[1]
    for b in range(n):
        mp = mp_ref[b]
        cv = candv_ref[b]
        cvT = jnp.transpose(cv)
        bestv = jnp.max(cvT, axis=1, keepdims=True)
        eq = cvT == bestv
        taken = jnp.zeros_like(bestv, dtype=jnp.bool_)
        xc = jnp.zeros_like(xcand_ref[0, 0])
        for q in range(qn):
            m = jnp.logical_and(eq[:, q:q + 1], jnp.logical_not(taken))
            taken = jnp.logical_or(taken, m)
            xc = xc + jnp.where(m, xcand_ref[b, q], 0.0)
        res = lax.dot_general(
            xc, mp, (((0,), (0,)), ((), ())),
            preferred_element_type=jnp.float32,
        )
        out_ref[b] = res[:c, :]


def _make_sc_argmax_gather(n, k, hw, cp):
    gpk = k // _L
    bpc = n // _NC
    qn = _NS // bpc
    ch = hw // qn

    mesh = plsc.VectorSubcoreMesh(
        core_axis_name="c",
        subcore_axis_name="s",
        num_cores=_NC,
        num_subcores=_NS,
    )

    @functools.partial(
        pl.kernel,
        mesh=mesh,
        out_type=[
            jax.ShapeDtypeStruct((n * qn, k), jnp.float32),
            jax.ShapeDtypeStruct((n * qn * k, cp), jnp.float32),
        ],
        scratch_types=[
            pltpu.VMEM((ch, k), jnp.float32),
            pltpu.VMEM((k,), jnp.float32),
            pltpu.VMEM((k,), jnp.int32),
            pltpu.VMEM((k, cp), jnp.float32),
            pltpu.SemaphoreType.DMA,
            pltpu.SemaphoreType.DMA,
        ],
    )
    def sc_fn(zt_hbm, xt_hbm, candv_hbm, xcand_hbm, z_v, candv_v, idx_v,
              rows_v, zsem, gsem):
        ci = lax.axis_index("c")
        s = lax.axis_index("s")
        b_local = s // qn
        batch = ci * bpc + b_local
        q = s % qn
        shard = batch * qn + q

        pltpu.async_copy(
            zt_hbm.at[batch, pl.ds(q * ch, ch)], z_v, zsem
        ).wait()

        def body(p, carry):
            new = []
            for g in range(gpk):
                vmax, vidx = carry[2 * g], carry[2 * g + 1]
                chunk = z_v[p, pl.ds(g * _L, _L)]
                upd = chunk > vmax
                vmax = jnp.where(upd, chunk, vmax)
                vidx = jnp.where(upd, p, vidx)
                new.extend((vmax, vidx))
            return tuple(new)

        init = []
        for _ in range(gpk):
            init.append(jnp.full((_L,), -jnp.inf, jnp.float32))
            init.append(jnp.zeros((_L,), jnp.int32))
        flat = lax.fori_loop(0, ch, body, tuple(init))

        for g in range(gpk):
            candv_v[pl.ds(g * _L, _L)] = flat[2 * g]
            idx_v[pl.ds(g * _L, _L)] = (
                flat[2 * g + 1] + q * ch + batch * hw
            )

        pltpu.async_copy(xt_hbm.at[idx_v], rows_v, gsem).wait()
        pltpu.sync_copy(rows_v, xcand_hbm.at[pl.ds(shard * k, k)])
        pltpu.sync_copy(candv_v, candv_hbm.at[shard])

    return sc_fn


def kernel(x, W, U):
    n, c, h, w = x.shape
    k = W.shape[0]
    hw = h * w
    cp = 128
    x2 = x.reshape(n, c, hw)
    U2t = U.reshape(n, k, hw).transpose(0, 2, 1)

    zt, xt, mp = pl.pallas_call(
        _stage_a_body,
        out_shape=[
            jax.ShapeDtypeStruct((n, hw, k), jnp.float32),
            jax.ShapeDtypeStruct((n, hw, cp), jnp.float32),
            jax.ShapeDtypeStruct((n, k, hw), jnp.float32),
        ],
    )(x2, W, U2t)

    qn = (_NC * _NS) // n
    sc_fn = _make_sc_argmax_gather(n, k, hw, cp)
    candv, xcand = sc_fn(zt, xt.reshape(n * hw, cp))

    out = pl.pallas_call(
        _stage_c_body,
        out_shape=jax.ShapeDtypeStruct((n, c, hw), jnp.float32),
    )(mp, candv.reshape(n, qn, k), xcand.reshape(n, qn, k, cp))
    return out.reshape(n, c, h, w)


# speedup vs baseline: 1.0053x; 1.0053x over previous
import functools

import jax
import jax.numpy as jnp
from jax import lax
from jax.experimental import pallas as pl
from jax.experimental.pallas import tpu as pltpu
from jax.experimental.pallas import tpu_sc as plsc

TOPK_NUM = 64
TEMP = 0.1
EPS = 1e-20

_NC = 2
_NS = 16
_L = 16


def _stage_a_body(x_ref, w_ref, ut_ref, zt_ref, xt_ref, mp_ref):
    W = w_ref[...]
    n, c, hw = x_ref.shape
    cp = xt_ref.shape[2]
    for b in range(n):
        x = x_ref[b]
        ut = ut_ref[b]
        xt = jnp.transpose(x)
        mt = lax.dot_general(
            xt, W, (((1,), (1,)), ((), ())),
            preferred_element_type=jnp.float32,
        )
        g = -jnp.log(-jnp.log(ut + EPS) + EPS)
        zt_ref[b] = mt + g
        xt_ref[b] = jnp.concatenate(
            [xt, jnp.zeros((hw, cp - c), jnp.float32)], axis=1
        )
        mmax = jnp.max(mt, axis=1, keepdims=True)
        e = jnp.exp(mt - mmax)
        mpT = e / jnp.sum(e, axis=1, keepdims=True)
        mp_ref[b] = jnp.transpose(mpT)


def _stage_c_body(mp_ref, packed_ref, out_ref):
    n, c, hw = out_ref.shape
    qn = packed_ref.shape[1]
    k = packed_ref.shape[2] - 8
    for b in range(n):
        mp = mp_ref[b]
        cv = packed_ref[b, :, k, :k]
        cvT = jnp.transpose(cv)
        bestv = jnp.max(cvT, axis=1, keepdims=True)
        eq = cvT == bestv
        taken = jnp.zeros_like(bestv, dtype=jnp.bool_)
        xc = jnp.zeros_like(packed_ref[0, 0, :k])
        for q in range(qn):
            m = jnp.logical_and(eq[:, q:q + 1], jnp.logical_not(taken))
            taken = jnp.logical_or(taken, m)
            xc = xc + jnp.where(m, packed_ref[b, q, :k], 0.0)
        res = lax.dot_general(
            xc, mp, (((0,), (0,)), ((), ())),
            preferred_element_type=jnp.float32,
        )
        out_ref[b] = res[:c, :]


def _make_sc_argmax_gather(n, k, hw, cp):
    gpk = k // _L
    bpc = n // _NC
    qn = _NS // bpc
    ch = hw // qn

    mesh = plsc.VectorSubcoreMesh(
        core_axis_name="c",
        subcore_axis_name="s",
        num_cores=_NC,
        num_subcores=_NS,
    )

    @functools.partial(
        pl.kernel,
        mesh=mesh,
        out_type=jax.ShapeDtypeStruct((n * qn * (k + 8), cp), jnp.float32),
        scratch_types=[
            pltpu.VMEM((ch, k), jnp.float32),
            pltpu.VMEM((cp,), jnp.float32),
            pltpu.VMEM((k,), jnp.int32),
            pltpu.VMEM((k, cp), jnp.float32),
            pltpu.SemaphoreType.DMA,
            pltpu.SemaphoreType.DMA,
        ],
    )
    def sc_fn(zt_hbm, xt_hbm, packed_hbm, z_v, candv_v, idx_v,
              rows_v, zsem, gsem):
        ci = lax.axis_index("c")
        s = lax.axis_index("s")
        b_local = s // qn
        batch = ci * bpc + b_local
        q = s % qn
        shard = batch * qn + q

        pltpu.async_copy(
            zt_hbm.at[batch, pl.ds(q * ch, ch)], z_v, zsem
        ).wait()

        def body(p, carry):
            new = []
            for g in range(gpk):
                vmax, vidx = carry[2 * g], carry[2 * g + 1]
                chunk = z_v[p, pl.ds(g * _L, _L)]
                upd = chunk > vmax
                vmax = jnp.where(upd, chunk, vmax)
                vidx = jnp.where(upd, p, vidx)
                new.extend((vmax, vidx))
            return tuple(new)

        init = []
        for _ in range(gpk):
            init.append(jnp.full((_L,), -jnp.inf, jnp.float32))
            init.append(jnp.zeros((_L,), jnp.int32))
        flat = lax.fori_loop(0, ch, body, tuple(init))

        for g in range(gpk):
            candv_v[pl.ds(g * _L, _L)] = flat[2 * g]
            idx_v[pl.ds(g * _L, _L)] = (
                flat[2 * g + 1] + q * ch + batch * hw
            )

        pltpu.async_copy(xt_hbm.at[idx_v], rows_v, gsem).wait()
        base = shard * (k + 8)
        pltpu.sync_copy(rows_v, packed_hbm.at[pl.ds(base, k)])
        pltpu.sync_copy(candv_v, packed_hbm.at[base + k])

    return sc_fn


def kernel(x, W, U):
    n, c, h, w = x.shape
    k = W.shape[0]
    hw = h * w
    cp = 128
    x2 = x.reshape(n, c, hw)
    U2t = U.reshape(n, k, hw).transpose(0, 2, 1)

    zt, xt, mp = pl.pallas_call(
        _stage_a_body,
        out_shape=[
            jax.ShapeDtypeStruct((n, hw, k), jnp.float32),
            jax.ShapeDtypeStruct((n, hw, cp), jnp.float32),
            jax.ShapeDtypeStruct((n, k, hw), jnp.float32),
        ],
    )(x2, W, U2t)

    qn = (_NC * _NS) // n
    sc_fn = _make_sc_argmax_gather(n, k, hw, cp)
    packed = sc_fn(zt, xt.reshape(n * hw, cp))

    out = pl.pallas_call(
        _stage_c_body,
        out_shape=jax.ShapeDtypeStruct((n, c, hw), jnp.float32),
    )(mp, packed.reshape(n, qn, k + 8, cp))
    return out.reshape(n, c, h, w)
